# baseline (speedup 1.0000x reference)
import functools

import jax
import jax.numpy as jnp
from jax.experimental import pallas as pl
from jax.experimental.pallas import tpu as pltpu

_BETA = 1.0 / 9
_NEG_POS_RATIO = 3
_LANES = 128
_BR = 512


def _imin():
    return jnp.int32(-2147483648)


def _imaxp():
    return jnp.int32(0x7FFFFFFF)


def _softplus(x):
    return jnp.maximum(x, 0.0) + jnp.log1p(jnp.exp(-jnp.abs(x)))


def _a_body(confb, labb, plb, glb, dc, rsum, v_out, acc):
    step = pl.program_id(0)
    d = jnp.dot(confb[...], dc[...], preferred_element_type=jnp.float32)
    bits = jax.lax.bitcast_convert_type(d, jnp.int32)
    v = jnp.where(bits >= 0, bits, bits ^ _imaxp())
    pos = labb[...] > 0
    v_out[...] = jnp.where(pos, _imin(), v)
    posf = pos.astype(jnp.float32)
    ce_pos = _softplus(-d)
    dl = jnp.abs(plb[...] - glb[...])
    sl = jnp.where(dl < _BETA, 0.5 * dl * dl / _BETA, dl - 0.5 * _BETA)
    vpa = jnp.dot(sl, rsum[...], preferred_element_type=jnp.float32)
    pcnt = jnp.sum(posf, axis=0)
    spos = jnp.sum(ce_pos * posf, axis=0)
    verp = jnp.sum(vpa * posf, axis=0)
    rows = jax.lax.broadcasted_iota(jnp.int32, (8, _LANES), 0)
    part = (
        jnp.where(rows == 0, pcnt[None, :], 0.0)
        + jnp.where(rows == 1, spos[None, :], 0.0)
        + jnp.where(rows == 2, verp[None, :], 0.0)
    )

    @pl.when(step == 0)
    def _():
        acc[...] = part

    @pl.when(step != 0)
    def _():
        acc[...] = acc[...] + part


def _b_body(n_total, v_ref, acc_ref, o_total, o_cls, o_ver):
    acc = acc_ref[...]
    npos_f = jnp.sum(acc[0, :])
    s_pos = jnp.sum(acc[1, :])
    ver_sum = jnp.sum(acc[2, :])
    npos = npos_f.astype(jnp.int32)
    k = jnp.minimum(npos * _NEG_POS_RATIO, jnp.int32(n_total) - npos)
    varr = v_ref[...]
    v3 = varr.reshape(16, 512, _LANES)

    def bs_body(i, t_u):
        cand = t_u | (jnp.int32(1) << (31 - i))
        cand_i = cand ^ _imin()
        part = jnp.sum((v3 >= cand_i).astype(jnp.int32), axis=1)
        cnt = jnp.sum(part)
        return jnp.where(cnt >= k, cand, t_u)

    t_u = jax.lax.fori_loop(0, 32, bs_body, jnp.int32(0), unroll=False)
    t_i = t_u ^ _imin()
    sel = varr > t_i
    eq = varr == t_i
    cnt_gt = jnp.sum(sel.astype(jnp.int32))
    u = varr ^ _imin()
    bits_f = jnp.where(u < 0, u & _imaxp(), jnp.bitwise_not(u))
    dd = jax.lax.bitcast_convert_type(bits_f, jnp.float32)
    ml = _softplus(dd)
    s_main = jnp.sum(jnp.where(sel, ml, 0.0))
    s_eq = jnp.sum(jnp.where(eq, ml, 0.0))
    c_eq = jnp.sum(eq.astype(jnp.float32))
    mlt = s_eq / c_eq
    s_sel = s_main + (k - cnt_gt).astype(jnp.float32) * mlt
    s_sel = jnp.where(k > 0, s_sel, 0.0)
    denom = jnp.maximum((npos + k).astype(jnp.float32), 1.0)
    cls = jnp.clip((s_pos + s_sel) / denom, 0.0, 5.0)
    ver = jnp.clip(ver_sum / jnp.maximum(2.0 * npos_f, 1.0), 0.0, 5.0)
    o_cls[0] = cls
    o_ver[0] = ver
    o_total[0] = cls + ver


def kernel(confidence, predicted_locations, labels, gt_locations):
    b, a = labels.shape
    n = b * a
    nr = n // _LANES
    grid = nr // _BR
    confr = confidence.reshape(nr, 2 * _LANES)
    plr = predicted_locations.reshape(nr, 4 * _LANES)
    glr = gt_locations.reshape(nr, 4 * _LANES)
    labr = labels.reshape(nr, _LANES)

    i2 = jnp.arange(2 * _LANES)[:, None]
    j = jnp.arange(_LANES)[None, :]
    dc = (i2 == 2 * j + 1).astype(jnp.float32) - (i2 == 2 * j).astype(jnp.float32)
    i4 = jnp.arange(4 * _LANES)[:, None]
    rsum = ((i4 == 4 * j + 1) | (i4 == 4 * j + 3)).astype(jnp.float32)

    smem_spec = pl.BlockSpec(memory_space=pltpu.SMEM)
    vmem_spec = pl.BlockSpec(memory_space=pltpu.VMEM)
    row_spec = pl.BlockSpec((_BR, _LANES), lambda i: (i, 0))
    acc_spec = pl.BlockSpec((8, _LANES), lambda i: (0, 0))
    v, acc = pl.pallas_call(
        _a_body,
        grid=(grid,),
        in_specs=[
            pl.BlockSpec((_BR, 2 * _LANES), lambda i: (i, 0)),
            pl.BlockSpec((_BR, _LANES), lambda i: (i, 0)),
            pl.BlockSpec((_BR, 4 * _LANES), lambda i: (i, 0)),
            pl.BlockSpec((_BR, 4 * _LANES), lambda i: (i, 0)),
            pl.BlockSpec((2 * _LANES, _LANES), lambda i: (0, 0)),
            pl.BlockSpec((4 * _LANES, _LANES), lambda i: (0, 0)),
        ],
        out_specs=[row_spec, acc_spec],
        out_shape=[
            jax.ShapeDtypeStruct((nr, _LANES), jnp.int32),
            jax.ShapeDtypeStruct((8, _LANES), jnp.float32),
        ],
    )(confr, labr, plr, glr, dc, rsum)

    total, cls, ver = pl.pallas_call(
        functools.partial(_b_body, n),
        in_specs=[vmem_spec, vmem_spec],
        out_specs=[smem_spec, smem_spec, smem_spec],
        out_shape=[
            jax.ShapeDtypeStruct((1,), jnp.float32),
            jax.ShapeDtypeStruct((1,), jnp.float32),
            jax.ShapeDtypeStruct((1,), jnp.float32),
        ],
    )(v, acc)

    z = jnp.zeros((), jnp.float32)
    return (total.reshape(()), cls.reshape(()), ver.reshape(()), z)


# speedup vs baseline: 44.8696x; 44.8696x over previous
import functools

import jax
import jax.numpy as jnp
from jax.experimental import pallas as pl
from jax.experimental.pallas import tpu as pltpu

_BETA = 1.0 / 9
_NEG_POS_RATIO = 3
_LANES = 128
_BR = 512
def _imin():
    return jnp.int32(-2147483648)


def _imaxp():
    return jnp.int32(0x7FFFFFFF)


def _softplus(x):
    return jnp.maximum(x, 0.0) + jnp.log1p(jnp.exp(-jnp.abs(x)))


def _a_body(c0, c1, lab, p1, p3, g1, g3, v_out, acc):
    step = pl.program_id(0)
    d = c1[...] - c0[...]
    bits = jax.lax.bitcast_convert_type(d, jnp.int32)
    v = jnp.where(bits >= 0, bits, bits ^ _imaxp())
    pos = lab[...] > 0
    v_out[...] = jnp.where(pos, _imin(), v)
    posf = pos.astype(jnp.float32)
    ce_pos = _softplus(-d)
    a1 = jnp.abs(p1[...] - g1[...])
    a3 = jnp.abs(p3[...] - g3[...])
    sl1 = jnp.where(a1 < _BETA, 0.5 * a1 * a1 / _BETA, a1 - 0.5 * _BETA)
    sl3 = jnp.where(a3 < _BETA, 0.5 * a3 * a3 / _BETA, a3 - 0.5 * _BETA)
    pcnt = jnp.sum(posf, axis=0)
    spos = jnp.sum(ce_pos * posf, axis=0)
    verp = jnp.sum((sl1 + sl3) * posf, axis=0)
    rows = jax.lax.broadcasted_iota(jnp.int32, (8, _LANES), 0)
    part = (
        jnp.where(rows == 0, pcnt[None, :], 0.0)
        + jnp.where(rows == 1, spos[None, :], 0.0)
        + jnp.where(rows == 2, verp[None, :], 0.0)
    )

    @pl.when(step == 0)
    def _():
        acc[...] = part

    @pl.when(step != 0)
    def _():
        acc[...] = acc[...] + part


def _b_body(n_total, v_ref, acc_ref, o_total, o_cls, o_ver):
    acc = acc_ref[...]
    npos_f = jnp.sum(acc[0, :])
    s_pos = jnp.sum(acc[1, :])
    ver_sum = jnp.sum(acc[2, :])
    npos = npos_f.astype(jnp.int32)
    k = jnp.minimum(npos * _NEG_POS_RATIO, jnp.int32(n_total) - npos)
    varr = v_ref[...]
    v3 = varr.reshape(16, 512, _LANES)

    def bs_body(i, t_u):
        cand = t_u | (jnp.int32(1) << (31 - i))
        cand_i = cand ^ _imin()
        part = jnp.sum((v3 >= cand_i).astype(jnp.int32), axis=1)
        cnt = jnp.sum(part)
        return jnp.where(cnt >= k, cand, t_u)

    t_u = jax.lax.fori_loop(0, 32, bs_body, jnp.int32(0), unroll=False)
    t_i = t_u ^ _imin()
    sel = varr > t_i
    eq = varr == t_i
    cnt_gt = jnp.sum(sel.astype(jnp.int32))
    u = varr ^ _imin()
    bits_f = jnp.where(u < 0, u & _imaxp(), jnp.bitwise_not(u))
    dd = jax.lax.bitcast_convert_type(bits_f, jnp.float32)
    ml = _softplus(dd)
    s_main = jnp.sum(jnp.where(sel, ml, 0.0))
    s_eq = jnp.sum(jnp.where(eq, ml, 0.0))
    c_eq = jnp.sum(eq.astype(jnp.float32))
    mlt = s_eq / c_eq
    s_sel = s_main + (k - cnt_gt).astype(jnp.float32) * mlt
    s_sel = jnp.where(k > 0, s_sel, 0.0)
    denom = jnp.maximum((npos + k).astype(jnp.float32), 1.0)
    cls = jnp.clip((s_pos + s_sel) / denom, 0.0, 5.0)
    ver = jnp.clip(ver_sum / jnp.maximum(2.0 * npos_f, 1.0), 0.0, 5.0)
    o_cls[0] = cls
    o_ver[0] = ver
    o_total[0] = cls + ver


def kernel(confidence, predicted_locations, labels, gt_locations):
    b, a = labels.shape
    n = b * a
    nr = n // _LANES
    grid = nr // _BR
    conf = confidence.reshape(n, 2)
    c0 = conf[:, 0].reshape(nr, _LANES)
    c1 = conf[:, 1].reshape(nr, _LANES)
    pl4 = predicted_locations.reshape(n, 4)
    gl4 = gt_locations.reshape(n, 4)
    p1 = pl4[:, 1].reshape(nr, _LANES)
    p3 = pl4[:, 3].reshape(nr, _LANES)
    g1 = gl4[:, 1].reshape(nr, _LANES)
    g3 = gl4[:, 3].reshape(nr, _LANES)
    lab = labels.reshape(nr, _LANES)

    row_spec = pl.BlockSpec((_BR, _LANES), lambda i: (i, 0))
    acc_spec = pl.BlockSpec((8, _LANES), lambda i: (0, 0))
    v, acc = pl.pallas_call(
        _a_body,
        grid=(grid,),
        in_specs=[row_spec] * 7,
        out_specs=[row_spec, acc_spec],
        out_shape=[
            jax.ShapeDtypeStruct((nr, _LANES), jnp.int32),
            jax.ShapeDtypeStruct((8, _LANES), jnp.float32),
        ],
    )(c0, c1, lab, p1, p3, g1, g3)

    total, cls, ver = pl.pallas_call(
        functools.partial(_b_body, n),
        in_specs=[
            pl.BlockSpec(memory_space=pltpu.VMEM),
            pl.BlockSpec(memory_space=pltpu.VMEM),
        ],
        out_specs=[
            pl.BlockSpec(memory_space=pltpu.SMEM),
            pl.BlockSpec(memory_space=pltpu.SMEM),
            pl.BlockSpec(memory_space=pltpu.SMEM),
        ],
        out_shape=[
            jax.ShapeDtypeStruct((1,), jnp.float32),
            jax.ShapeDtypeStruct((1,), jnp.float32),
            jax.ShapeDtypeStruct((1,), jnp.float32),
        ],
    )(v, acc)

    z = jnp.zeros((), jnp.float32)
    return (total.reshape(()), cls.reshape(()), ver.reshape(()), z)
